# baseline (speedup 1.0000x reference)
import functools

import jax
import jax.numpy as jnp
from jax import lax
from jax.experimental import pallas as pl
from jax.experimental.pallas import tpu as pltpu
from jax.experimental.pallas import tpu_sc as plsc

F32 = jnp.float32
I32 = jnp.int32

VOCAB = 30000
EMB = 128
D = 768
FFN = 3072
E = 8
B = 4
S = 512
N = B * S
CAP = 512
EPS = 1e-12
DISP_ROWS = E * CAP + 8
DUMMY_ROW = E * CAP

NC, NS = 2, 16
NW = NC * NS
TPW = N // NW


@functools.lru_cache(maxsize=None)
def _make_sc_row_gather(n_rows_out, d):
    per_w = n_rows_out // NW
    mesh = plsc.VectorSubcoreMesh(core_axis_name="c", subcore_axis_name="s")

    @functools.partial(
        pl.kernel,
        out_type=jax.ShapeDtypeStruct((n_rows_out, d), F32),
        mesh=mesh,
        scratch_types=[
            pltpu.VMEM((per_w,), I32),
            pltpu.VMEM((per_w, d), F32),
            pltpu.SemaphoreType.DMA,
        ],
    )
    def k(table_hbm, idx_hbm, out_hbm, idx_v, rows_v, sem):
        wid = lax.axis_index("s") * NC + lax.axis_index("c")
        base = wid * per_w
        pltpu.sync_copy(idx_hbm.at[pl.ds(base, per_w)], idx_v)
        pltpu.async_copy(table_hbm.at[idx_v], rows_v, sem).wait()
        pltpu.sync_copy(rows_v, out_hbm.at[pl.ds(base, per_w)])

    return k


@functools.lru_cache(maxsize=None)
def _make_sc_row_scatter(n_rows_in, d, out_rows):
    per_w = n_rows_in // NW
    mesh = plsc.VectorSubcoreMesh(core_axis_name="c", subcore_axis_name="s")

    @functools.partial(
        pl.kernel,
        out_type=jax.ShapeDtypeStruct((out_rows, d), F32),
        mesh=mesh,
        scratch_types=[
            pltpu.VMEM((per_w,), I32),
            pltpu.VMEM((per_w, d), F32),
            pltpu.SemaphoreType.DMA,
        ],
    )
    def k(x_hbm, dest_hbm, out_hbm, dest_v, rows_v, sem):
        wid = lax.axis_index("s") * NC + lax.axis_index("c")
        base = wid * per_w
        pltpu.sync_copy(dest_hbm.at[pl.ds(base, per_w)], dest_v)
        pltpu.sync_copy(x_hbm.at[pl.ds(base, per_w)], rows_v)
        pltpu.async_copy(rows_v, out_hbm.at[dest_v], sem).wait()

    return k


def _sc_embed_gather(table, ids):
    return _make_sc_row_gather(N, EMB)(table, ids)


def _sc_dispatch(x, dest):
    return _make_sc_row_scatter(N, D, DISP_ROWS)(x, dest)


def _sc_combine(eout, gidx):
    return _make_sc_row_gather(N, D)(eout, gidx)


TB_PRE = 512
NB_PRE = N // TB_PRE


def _tc_pre_body(emb_ref, pos_ref, typ_ref, lng_ref, lnb_ref, mapw_ref,
                 mapb_ref, gatew_ref, gateb_ref,
                 x_ref, dest_ref, gidx_ref, p_ref, keep_ref, bal_ref,
                 carry, facc, pacc):
    i = pl.program_id(0)

    @pl.when(i == 0)
    def _():
        carry[...] = jnp.zeros_like(carry)
        facc[...] = jnp.zeros_like(facc)
        pacc[...] = jnp.zeros_like(pacc)

    e = emb_ref[...] + pos_ref[...] + typ_ref[...]
    mu = jnp.mean(e, axis=-1, keepdims=True)
    var = jnp.mean((e - mu) ** 2, axis=-1, keepdims=True)
    eln = (e - mu) / jnp.sqrt(var + EPS) * lng_ref[...] + lnb_ref[...]
    x = jnp.dot(eln, mapw_ref[...], preferred_element_type=F32) + mapb_ref[...]
    x_ref[...] = x

    logits = jnp.dot(x, gatew_ref[...], preferred_element_type=F32) + gateb_ref[...]
    mx = jnp.max(logits, axis=-1, keepdims=True)
    ex = jnp.exp(logits - mx)
    probs = ex / jnp.sum(ex, axis=-1, keepdims=True)
    eids = lax.broadcasted_iota(I32, (TB_PRE, E), 1)
    idxv = jnp.min(jnp.where(logits >= mx, eids, E), axis=-1, keepdims=True)
    onehot = (eids == idxv).astype(F32)
    pmax = jnp.max(probs, axis=-1, keepdims=True)

    r = lax.broadcasted_iota(I32, (TB_PRE, TB_PRE), 0)
    c = lax.broadcasted_iota(I32, (TB_PRE, TB_PRE), 1)
    tri = (r >= c).astype(F32)
    pos_cum = jnp.dot(tri, onehot, preferred_element_type=F32) + carry[...]
    carry[...] = carry[...] + jnp.sum(onehot, axis=0, keepdims=True)

    slot = jnp.sum(onehot * pos_cum, axis=-1, keepdims=True).astype(I32) - 1
    keep = slot < CAP
    rowid = idxv * CAP + slot
    dest_ref[...] = jnp.where(keep, rowid, DUMMY_ROW)
    gidx_ref[...] = jnp.where(keep, rowid, 0)
    p_ref[...] = pmax
    keep_ref[...] = keep.astype(F32)

    facc[...] = facc[...] + jnp.sum(onehot, axis=0, keepdims=True)
    pacc[...] = pacc[...] + jnp.sum(probs, axis=0, keepdims=True)

    @pl.when(i == NB_PRE - 1)
    def _():
        f = facc[...] / float(N)
        pmean = pacc[...] / float(N)
        bal_ref[0, 0] = jnp.sum(f * pmean) * float(E)


def _tc_pre(emb_rows, pos, typ, lng, lnb, mapw, mapb, gatew, gateb):
    full2 = lambda shape: pl.BlockSpec(shape, lambda i: (0, 0))
    return pl.pallas_call(
        _tc_pre_body,
        grid=(NB_PRE,),
        in_specs=[
            pl.BlockSpec((TB_PRE, EMB), lambda i: (i, 0)),
            full2((S, EMB)),
            full2((1, EMB)),
            full2((1, EMB)),
            full2((1, EMB)),
            full2((EMB, D)),
            full2((1, D)),
            full2((D, E)),
            full2((1, E)),
        ],
        out_specs=[
            pl.BlockSpec((TB_PRE, D), lambda i: (i, 0)),
            pl.BlockSpec((TB_PRE, 1), lambda i: (i, 0)),
            pl.BlockSpec((TB_PRE, 1), lambda i: (i, 0)),
            pl.BlockSpec((TB_PRE, 1), lambda i: (i, 0)),
            pl.BlockSpec((TB_PRE, 1), lambda i: (i, 0)),
            pl.BlockSpec(memory_space=pltpu.SMEM),
        ],
        out_shape=[
            jax.ShapeDtypeStruct((N, D), F32),
            jax.ShapeDtypeStruct((N, 1), I32),
            jax.ShapeDtypeStruct((N, 1), I32),
            jax.ShapeDtypeStruct((N, 1), F32),
            jax.ShapeDtypeStruct((N, 1), F32),
            jax.ShapeDtypeStruct((1, 1), F32),
        ],
        scratch_shapes=[
            pltpu.VMEM((1, E), F32),
            pltpu.VMEM((1, E), F32),
            pltpu.VMEM((1, E), F32),
        ],
    )(emb_rows, pos, typ, lng, lnb, mapw, mapb, gatew, gateb)


def _tc_experts_body(disp_ref, w1_ref, b1_ref, w2_ref, b2_ref, out_ref):
    d = disp_ref[...].astype(jnp.bfloat16)
    h = jnp.dot(d, w1_ref[0], preferred_element_type=F32) + b1_ref[0]
    h = jax.nn.gelu(h)
    out = jnp.dot(h.astype(jnp.bfloat16), w2_ref[0],
                  preferred_element_type=F32) + b2_ref[0]
    out_ref[...] = out


def _tc_experts(disp, w1, b1, w2, b2):
    return pl.pallas_call(
        _tc_experts_body,
        grid=(E,),
        in_specs=[
            pl.BlockSpec((CAP, D), lambda e: (e, 0)),
            pl.BlockSpec((1, D, FFN), lambda e: (e, 0, 0)),
            pl.BlockSpec((1, 1, FFN), lambda e: (e, 0, 0)),
            pl.BlockSpec((1, FFN, D), lambda e: (e, 0, 0)),
            pl.BlockSpec((1, 1, D), lambda e: (e, 0, 0)),
        ],
        out_specs=pl.BlockSpec((CAP, D), lambda e: (e, 0)),
        out_shape=jax.ShapeDtypeStruct((E * CAP, D), F32),
    )(disp, w1, b1.reshape(E, 1, FFN), w2, b2.reshape(E, 1, D))


TB_MLM = 64
NB_MLM = N // TB_MLM
BPB = S // TB_MLM


def _tc_mlm_body(y_ref, p_ref, keep_ref, lbl_ref, mlmw_ref, mlmb_ref,
                 lng_ref, lnb_ref, decw_ref, decb_ref, bal_ref,
                 h_ref, sc_ref, loss_ref, lacc):
    i = pl.program_id(0)

    @pl.when(i == 0)
    def _():
        lacc[0, 0] = 0.0

    y = jnp.where(keep_ref[...] > 0.0, y_ref[...] * p_ref[...], 0.0)
    h_ref[...] = y

    m = jnp.dot(y, mlmw_ref[...], preferred_element_type=F32) + mlmb_ref[...]
    m = jax.nn.gelu(m)
    mu = jnp.mean(m, axis=-1, keepdims=True)
    var = jnp.mean((m - mu) ** 2, axis=-1, keepdims=True)
    mln = (m - mu) / jnp.sqrt(var + EPS) * lng_ref[...] + lnb_ref[...]

    sc = jnp.dot(mln.astype(jnp.bfloat16), decw_ref[...],
                 preferred_element_type=F32) + decb_ref[...]
    sc_ref[...] = sc

    mx = jnp.max(sc, axis=-1, keepdims=True)
    lse = mx + jnp.log(jnp.sum(jnp.exp(sc - mx), axis=-1, keepdims=True))
    vid = lax.broadcasted_iota(I32, (TB_MLM, VOCAB), 1)
    slab = jnp.sum(jnp.where(vid == lbl_ref[...], sc, 0.0), axis=-1,
                   keepdims=True)
    lacc[0, 0] += jnp.sum(lse - slab)

    @pl.when(i == NB_MLM - 1)
    def _():
        loss_ref[0, 0] = lacc[0, 0] / float(N) + bal_ref[0, 0]


def _tc_mlm(yraw, p, keep, lbl, mlmw, mlmb, lng, lnb, decw, decb, bal):
    full2 = lambda shape: pl.BlockSpec(shape, lambda i: (0, 0))
    tok1 = lambda dt: (pl.BlockSpec((TB_MLM, 1), lambda i: (i, 0)),
                       jax.ShapeDtypeStruct((N, 1), dt))
    return pl.pallas_call(
        _tc_mlm_body,
        grid=(NB_MLM,),
        in_specs=[
            pl.BlockSpec((TB_MLM, D), lambda i: (i, 0)),
            pl.BlockSpec((TB_MLM, 1), lambda i: (i, 0)),
            pl.BlockSpec((TB_MLM, 1), lambda i: (i, 0)),
            pl.BlockSpec((TB_MLM, 1), lambda i: (i, 0)),
            full2((D, EMB)),
            full2((1, EMB)),
            full2((1, EMB)),
            full2((1, EMB)),
            full2((EMB, VOCAB)),
            full2((1, VOCAB)),
            pl.BlockSpec(memory_space=pltpu.SMEM),
        ],
        out_specs=[
            pl.BlockSpec((TB_MLM, D), lambda i: (i, 0)),
            pl.BlockSpec((TB_MLM, VOCAB), lambda i: (i, 0)),
            pl.BlockSpec(memory_space=pltpu.SMEM),
        ],
        out_shape=[
            jax.ShapeDtypeStruct((N, D), F32),
            jax.ShapeDtypeStruct((N, VOCAB), F32),
            jax.ShapeDtypeStruct((1, 1), F32),
        ],
        scratch_shapes=[pltpu.SMEM((1, 1), F32)],
    )(yraw, p, keep, lbl, mlmw, mlmb, lng, lnb, decw, decb, bal)


def kernel(input_ids, labels, word_emb, pos_emb, type_emb, emb_ln_g, emb_ln_b,
           map_W, map_b, gate_W, gate_b, w1, b1, w2, b2,
           mlm_W, mlm_b, mlm_ln_g, mlm_ln_b, dec_W, dec_b):
    ids = input_ids.reshape(-1).astype(I32)

    emb_rows = _sc_embed_gather(word_emb, ids)

    x, dest, gidx, pval, keep, bal = _tc_pre(
        emb_rows, pos_emb[:S], type_emb[0:1],
        emb_ln_g.reshape(1, EMB), emb_ln_b.reshape(1, EMB),
        map_W, map_b.reshape(1, D), gate_W, gate_b.reshape(1, E))

    disp = _sc_dispatch(x, dest.reshape(-1))
    eout = _tc_experts(disp, w1.astype(jnp.bfloat16), b1,
                       w2.astype(jnp.bfloat16), b2)
    yraw = _sc_combine(eout, gidx.reshape(-1))

    h, scores, loss = _tc_mlm(
        yraw, pval, keep, labels.reshape(-1, 1).astype(I32),
        mlm_W, mlm_b.reshape(1, EMB),
        mlm_ln_g.reshape(1, EMB), mlm_ln_b.reshape(1, EMB),
        dec_W.astype(jnp.bfloat16), dec_b.reshape(1, VOCAB), bal)

    return (loss[0, 0], scores.reshape(B, S, VOCAB), h.reshape(B, S, D))


# speedup vs baseline: 1.1123x; 1.1123x over previous
import functools

import jax
import jax.numpy as jnp
from jax import lax
from jax.experimental import pallas as pl
from jax.experimental.pallas import tpu as pltpu
from jax.experimental.pallas import tpu_sc as plsc

F32 = jnp.float32
I32 = jnp.int32

VOCAB = 30000
EMB = 128
D = 768
FFN = 3072
E = 8
B = 4
S = 512
N = B * S
CAP = 512
EPS = 1e-12
DISP_ROWS = E * CAP + 8
DUMMY_ROW = E * CAP

NC, NS = 2, 16
NW = NC * NS
TPW = N // NW


@functools.lru_cache(maxsize=None)
def _make_sc_row_gather(n_rows_out, d):
    per_w = n_rows_out // NW
    mesh = plsc.VectorSubcoreMesh(core_axis_name="c", subcore_axis_name="s")

    @functools.partial(
        pl.kernel,
        out_type=jax.ShapeDtypeStruct((n_rows_out, d), F32),
        mesh=mesh,
        scratch_types=[
            pltpu.VMEM((per_w,), I32),
            pltpu.VMEM((per_w, d), F32),
            pltpu.SemaphoreType.DMA,
        ],
    )
    def k(table_hbm, idx_hbm, out_hbm, idx_v, rows_v, sem):
        wid = lax.axis_index("s") * NC + lax.axis_index("c")
        base = wid * per_w
        pltpu.sync_copy(idx_hbm.at[pl.ds(base, per_w)], idx_v)
        pltpu.async_copy(table_hbm.at[idx_v], rows_v, sem).wait()
        pltpu.sync_copy(rows_v, out_hbm.at[pl.ds(base, per_w)])

    return k


@functools.lru_cache(maxsize=None)
def _make_sc_row_scatter(n_rows_in, d, out_rows):
    per_w = n_rows_in // NW
    mesh = plsc.VectorSubcoreMesh(core_axis_name="c", subcore_axis_name="s")

    @functools.partial(
        pl.kernel,
        out_type=jax.ShapeDtypeStruct((out_rows, d), F32),
        mesh=mesh,
        scratch_types=[
            pltpu.VMEM((per_w,), I32),
            pltpu.VMEM((per_w, d), F32),
            pltpu.SemaphoreType.DMA,
        ],
    )
    def k(x_hbm, dest_hbm, out_hbm, dest_v, rows_v, sem):
        wid = lax.axis_index("s") * NC + lax.axis_index("c")
        base = wid * per_w
        pltpu.sync_copy(dest_hbm.at[pl.ds(base, per_w)], dest_v)
        pltpu.sync_copy(x_hbm.at[pl.ds(base, per_w)], rows_v)
        pltpu.async_copy(rows_v, out_hbm.at[dest_v], sem).wait()

    return k


def _sc_embed_gather(table, ids):
    return _make_sc_row_gather(N, EMB)(table, ids)


def _sc_dispatch(x, dest):
    return _make_sc_row_scatter(N, D, DISP_ROWS)(x, dest)


def _sc_combine(eout, gidx):
    return _make_sc_row_gather(N, D)(eout, gidx)


TB_PRE = 512
NB_PRE = N // TB_PRE


def _tc_pre_body(emb_ref, pos_ref, typ_ref, lng_ref, lnb_ref, mapw_ref,
                 mapb_ref, gatew_ref, gateb_ref,
                 x_ref, dest_ref, gidx_ref, p_ref, keep_ref, bal_ref,
                 carry, facc, pacc):
    i = pl.program_id(0)

    @pl.when(i == 0)
    def _():
        carry[...] = jnp.zeros_like(carry)
        facc[...] = jnp.zeros_like(facc)
        pacc[...] = jnp.zeros_like(pacc)

    e = emb_ref[...] + pos_ref[...] + typ_ref[...]
    mu = jnp.mean(e, axis=-1, keepdims=True)
    var = jnp.mean((e - mu) ** 2, axis=-1, keepdims=True)
    eln = (e - mu) / jnp.sqrt(var + EPS) * lng_ref[...] + lnb_ref[...]
    x = jnp.dot(eln, mapw_ref[...], preferred_element_type=F32) + mapb_ref[...]
    x_ref[...] = x

    logits = jnp.dot(x, gatew_ref[...], preferred_element_type=F32) + gateb_ref[...]
    mx = jnp.max(logits, axis=-1, keepdims=True)
    ex = jnp.exp(logits - mx)
    probs = ex / jnp.sum(ex, axis=-1, keepdims=True)
    eids = lax.broadcasted_iota(I32, (TB_PRE, E), 1)
    idxv = jnp.min(jnp.where(logits >= mx, eids, E), axis=-1, keepdims=True)
    onehot = (eids == idxv).astype(F32)
    pmax = jnp.max(probs, axis=-1, keepdims=True)

    r = lax.broadcasted_iota(I32, (TB_PRE, TB_PRE), 0)
    c = lax.broadcasted_iota(I32, (TB_PRE, TB_PRE), 1)
    tri = (r >= c).astype(F32)
    pos_cum = jnp.dot(tri, onehot, preferred_element_type=F32) + carry[...]
    carry[...] = carry[...] + jnp.sum(onehot, axis=0, keepdims=True)

    slot = jnp.sum(onehot * pos_cum, axis=-1, keepdims=True).astype(I32) - 1
    keep = slot < CAP
    rowid = idxv * CAP + slot
    dest_ref[...] = jnp.where(keep, rowid, DUMMY_ROW)
    gidx_ref[...] = jnp.where(keep, rowid, 0)
    p_ref[...] = pmax
    keep_ref[...] = keep.astype(F32)

    facc[...] = facc[...] + jnp.sum(onehot, axis=0, keepdims=True)
    pacc[...] = pacc[...] + jnp.sum(probs, axis=0, keepdims=True)

    @pl.when(i == NB_PRE - 1)
    def _():
        f = facc[...] / float(N)
        pmean = pacc[...] / float(N)
        bal_ref[0, 0] = jnp.sum(f * pmean) * float(E)


def _tc_pre(emb_rows, pos, typ, lng, lnb, mapw, mapb, gatew, gateb):
    full2 = lambda shape: pl.BlockSpec(shape, lambda i: (0, 0))
    return pl.pallas_call(
        _tc_pre_body,
        grid=(NB_PRE,),
        in_specs=[
            pl.BlockSpec((TB_PRE, EMB), lambda i: (i, 0)),
            full2((S, EMB)),
            full2((1, EMB)),
            full2((1, EMB)),
            full2((1, EMB)),
            full2((EMB, D)),
            full2((1, D)),
            full2((D, E)),
            full2((1, E)),
        ],
        out_specs=[
            pl.BlockSpec((TB_PRE, D), lambda i: (i, 0)),
            pl.BlockSpec((TB_PRE, 1), lambda i: (i, 0)),
            pl.BlockSpec((TB_PRE, 1), lambda i: (i, 0)),
            pl.BlockSpec((TB_PRE, 1), lambda i: (i, 0)),
            pl.BlockSpec((TB_PRE, 1), lambda i: (i, 0)),
            pl.BlockSpec(memory_space=pltpu.SMEM),
        ],
        out_shape=[
            jax.ShapeDtypeStruct((N, D), F32),
            jax.ShapeDtypeStruct((N, 1), I32),
            jax.ShapeDtypeStruct((N, 1), I32),
            jax.ShapeDtypeStruct((N, 1), F32),
            jax.ShapeDtypeStruct((N, 1), F32),
            jax.ShapeDtypeStruct((1, 1), F32),
        ],
        scratch_shapes=[
            pltpu.VMEM((1, E), F32),
            pltpu.VMEM((1, E), F32),
            pltpu.VMEM((1, E), F32),
        ],
    )(emb_rows, pos, typ, lng, lnb, mapw, mapb, gatew, gateb)


def _tc_experts_body(disp_ref, w1_ref, b1_ref, w2_ref, b2_ref, out_ref):
    d = disp_ref[...].astype(jnp.bfloat16)
    h = jnp.dot(d, w1_ref[0].astype(jnp.bfloat16),
                preferred_element_type=F32) + b1_ref[0]
    h = jax.nn.gelu(h)
    out = jnp.dot(h.astype(jnp.bfloat16), w2_ref[0].astype(jnp.bfloat16),
                  preferred_element_type=F32) + b2_ref[0]
    out_ref[...] = out


def _tc_experts(disp, w1, b1, w2, b2):
    return pl.pallas_call(
        _tc_experts_body,
        grid=(E,),
        in_specs=[
            pl.BlockSpec((CAP, D), lambda e: (e, 0)),
            pl.BlockSpec((1, D, FFN), lambda e: (e, 0, 0)),
            pl.BlockSpec((1, 1, FFN), lambda e: (e, 0, 0)),
            pl.BlockSpec((1, FFN, D), lambda e: (e, 0, 0)),
            pl.BlockSpec((1, 1, D), lambda e: (e, 0, 0)),
        ],
        out_specs=pl.BlockSpec((CAP, D), lambda e: (e, 0)),
        out_shape=jax.ShapeDtypeStruct((E * CAP, D), F32),
    )(disp, w1, b1.reshape(E, 1, FFN), w2, b2.reshape(E, 1, D))


TB_MLM = 64
NB_MLM = N // TB_MLM
BPB = S // TB_MLM


def _tc_mlm_body(y_ref, p_ref, keep_ref, lbl_ref, mlmw_ref, mlmb_ref,
                 lng_ref, lnb_ref, decw_ref, decb_ref, bal_ref,
                 h_ref, sc_ref, loss_ref, lacc):
    i = pl.program_id(0)

    @pl.when(i == 0)
    def _():
        lacc[0, 0] = 0.0

    y = jnp.where(keep_ref[...] > 0.0, y_ref[...] * p_ref[...], 0.0)
    h_ref[...] = y

    m = jnp.dot(y, mlmw_ref[...], preferred_element_type=F32) + mlmb_ref[...]
    m = jax.nn.gelu(m)
    mu = jnp.mean(m, axis=-1, keepdims=True)
    var = jnp.mean((m - mu) ** 2, axis=-1, keepdims=True)
    mln = (m - mu) / jnp.sqrt(var + EPS) * lng_ref[...] + lnb_ref[...]

    sc = jnp.dot(mln.astype(jnp.bfloat16), decw_ref[...],
                 preferred_element_type=F32) + decb_ref[...]
    sc_ref[...] = sc

    mx = jnp.max(sc, axis=-1, keepdims=True)
    lse = mx + jnp.log(jnp.sum(jnp.exp(sc - mx), axis=-1, keepdims=True))
    vid = lax.broadcasted_iota(I32, (TB_MLM, VOCAB), 1)
    slab = jnp.sum(jnp.where(vid == lbl_ref[...], sc, 0.0), axis=-1,
                   keepdims=True)
    lacc[0, 0] += jnp.sum(lse - slab)

    @pl.when(i == NB_MLM - 1)
    def _():
        loss_ref[0, 0] = lacc[0, 0] / float(N) + bal_ref[0, 0]


def _tc_mlm(yraw, p, keep, lbl, mlmw, mlmb, lng, lnb, decw, decb, bal):
    full2 = lambda shape: pl.BlockSpec(shape, lambda i: (0, 0))
    tok1 = lambda dt: (pl.BlockSpec((TB_MLM, 1), lambda i: (i, 0)),
                       jax.ShapeDtypeStruct((N, 1), dt))
    return pl.pallas_call(
        _tc_mlm_body,
        grid=(NB_MLM,),
        in_specs=[
            pl.BlockSpec((TB_MLM, D), lambda i: (i, 0)),
            pl.BlockSpec((TB_MLM, 1), lambda i: (i, 0)),
            pl.BlockSpec((TB_MLM, 1), lambda i: (i, 0)),
            pl.BlockSpec((TB_MLM, 1), lambda i: (i, 0)),
            full2((D, EMB)),
            full2((1, EMB)),
            full2((1, EMB)),
            full2((1, EMB)),
            full2((EMB, VOCAB)),
            full2((1, VOCAB)),
            pl.BlockSpec(memory_space=pltpu.SMEM),
        ],
        out_specs=[
            pl.BlockSpec((TB_MLM, D), lambda i: (i, 0)),
            pl.BlockSpec((TB_MLM, VOCAB), lambda i: (i, 0)),
            pl.BlockSpec(memory_space=pltpu.SMEM),
        ],
        out_shape=[
            jax.ShapeDtypeStruct((N, D), F32),
            jax.ShapeDtypeStruct((N, VOCAB), F32),
            jax.ShapeDtypeStruct((1, 1), F32),
        ],
        scratch_shapes=[pltpu.SMEM((1, 1), F32)],
    )(yraw, p, keep, lbl, mlmw, mlmb, lng, lnb, decw, decb, bal)


def kernel(input_ids, labels, word_emb, pos_emb, type_emb, emb_ln_g, emb_ln_b,
           map_W, map_b, gate_W, gate_b, w1, b1, w2, b2,
           mlm_W, mlm_b, mlm_ln_g, mlm_ln_b, dec_W, dec_b):
    ids = input_ids.reshape(-1).astype(I32)

    emb_rows = _sc_embed_gather(word_emb, ids)

    x, dest, gidx, pval, keep, bal = _tc_pre(
        emb_rows, pos_emb[:S], type_emb[0:1],
        emb_ln_g.reshape(1, EMB), emb_ln_b.reshape(1, EMB),
        map_W, map_b.reshape(1, D), gate_W, gate_b.reshape(1, E))

    disp = _sc_dispatch(x, dest.reshape(-1))
    eout = _tc_experts(disp, w1, b1, w2, b2)
    yraw = _sc_combine(eout, gidx.reshape(-1))

    h, scores, loss = _tc_mlm(
        yraw, pval, keep, labels.reshape(-1, 1).astype(I32),
        mlm_W, mlm_b.reshape(1, EMB),
        mlm_ln_g.reshape(1, EMB), mlm_ln_b.reshape(1, EMB),
        dec_W.astype(jnp.bfloat16), dec_b.reshape(1, VOCAB), bal)

    return (loss[0, 0], scores.reshape(B, S, VOCAB), h.reshape(B, S, D))


# speedup vs baseline: 1.1361x; 1.0214x over previous
import functools

import jax
import jax.numpy as jnp
from jax import lax
from jax.experimental import pallas as pl
from jax.experimental.pallas import tpu as pltpu
from jax.experimental.pallas import tpu_sc as plsc

F32 = jnp.float32
I32 = jnp.int32

VOCAB = 30000
EMB = 128
D = 768
FFN = 3072
E = 8
B = 4
S = 512
N = B * S
CAP = 512
EPS = 1e-12
DISP_ROWS = E * CAP + 8
DUMMY_ROW = E * CAP

NC, NS = 2, 16
NW = NC * NS
TPW = N // NW


@functools.lru_cache(maxsize=None)
def _make_sc_row_gather(n_rows_out, d):
    per_w = n_rows_out // NW
    mesh = plsc.VectorSubcoreMesh(core_axis_name="c", subcore_axis_name="s")

    @functools.partial(
        pl.kernel,
        out_type=jax.ShapeDtypeStruct((n_rows_out, d), F32),
        mesh=mesh,
        scratch_types=[
            pltpu.VMEM((per_w,), I32),
            pltpu.VMEM((per_w, d), F32),
            pltpu.SemaphoreType.DMA,
        ],
    )
    def k(table_hbm, idx_hbm, out_hbm, idx_v, rows_v, sem):
        wid = lax.axis_index("s") * NC + lax.axis_index("c")
        base = wid * per_w
        pltpu.sync_copy(idx_hbm.at[pl.ds(base, per_w)], idx_v)
        pltpu.async_copy(table_hbm.at[idx_v], rows_v, sem).wait()
        pltpu.sync_copy(rows_v, out_hbm.at[pl.ds(base, per_w)])

    return k


@functools.lru_cache(maxsize=None)
def _make_sc_row_scatter(n_rows_in, d, out_rows):
    per_w = n_rows_in // NW
    mesh = plsc.VectorSubcoreMesh(core_axis_name="c", subcore_axis_name="s")

    @functools.partial(
        pl.kernel,
        out_type=jax.ShapeDtypeStruct((out_rows, d), F32),
        mesh=mesh,
        scratch_types=[
            pltpu.VMEM((per_w,), I32),
            pltpu.VMEM((per_w, d), F32),
            pltpu.SemaphoreType.DMA,
        ],
    )
    def k(x_hbm, dest_hbm, out_hbm, dest_v, rows_v, sem):
        wid = lax.axis_index("s") * NC + lax.axis_index("c")
        base = wid * per_w
        pltpu.sync_copy(dest_hbm.at[pl.ds(base, per_w)], dest_v)
        pltpu.sync_copy(x_hbm.at[pl.ds(base, per_w)], rows_v)
        pltpu.async_copy(rows_v, out_hbm.at[dest_v], sem).wait()

    return k


def _sc_embed_gather(table, ids):
    return _make_sc_row_gather(N, EMB)(table, ids)


def _sc_dispatch(x, dest):
    return _make_sc_row_scatter(N, D, DISP_ROWS)(x, dest)


def _sc_combine(eout, gidx):
    return _make_sc_row_gather(N, D)(eout, gidx)


TB_PRE = 512
NB_PRE = N // TB_PRE


def _tc_pre_body(emb_ref, pos_ref, typ_ref, lng_ref, lnb_ref, mapw_ref,
                 mapb_ref, gatew_ref, gateb_ref,
                 x_ref, dest_ref, gidx_ref, p_ref, keep_ref, bal_ref,
                 carry, facc, pacc):
    i = pl.program_id(0)

    @pl.when(i == 0)
    def _():
        carry[...] = jnp.zeros_like(carry)
        facc[...] = jnp.zeros_like(facc)
        pacc[...] = jnp.zeros_like(pacc)

    e = emb_ref[...] + pos_ref[...] + typ_ref[...]
    mu = jnp.mean(e, axis=-1, keepdims=True)
    var = jnp.mean((e - mu) ** 2, axis=-1, keepdims=True)
    eln = (e - mu) / jnp.sqrt(var + EPS) * lng_ref[...] + lnb_ref[...]
    x = jnp.dot(eln, mapw_ref[...], preferred_element_type=F32) + mapb_ref[...]
    x_ref[...] = x

    logits = jnp.dot(x, gatew_ref[...], preferred_element_type=F32) + gateb_ref[...]
    mx = jnp.max(logits, axis=-1, keepdims=True)
    ex = jnp.exp(logits - mx)
    probs = ex / jnp.sum(ex, axis=-1, keepdims=True)
    eids = lax.broadcasted_iota(I32, (TB_PRE, E), 1)
    idxv = jnp.min(jnp.where(logits >= mx, eids, E), axis=-1, keepdims=True)
    onehot = (eids == idxv).astype(F32)
    pmax = jnp.max(probs, axis=-1, keepdims=True)

    r = lax.broadcasted_iota(I32, (TB_PRE, TB_PRE), 0)
    c = lax.broadcasted_iota(I32, (TB_PRE, TB_PRE), 1)
    tri = (r >= c).astype(F32)
    pos_cum = jnp.dot(tri, onehot, preferred_element_type=F32) + carry[...]
    carry[...] = carry[...] + jnp.sum(onehot, axis=0, keepdims=True)

    slot = jnp.sum(onehot * pos_cum, axis=-1, keepdims=True).astype(I32) - 1
    keep = slot < CAP
    rowid = idxv * CAP + slot
    dest_ref[...] = jnp.where(keep, rowid, DUMMY_ROW)
    gidx_ref[...] = jnp.where(keep, rowid, 0)
    p_ref[...] = pmax
    keep_ref[...] = keep.astype(F32)

    facc[...] = facc[...] + jnp.sum(onehot, axis=0, keepdims=True)
    pacc[...] = pacc[...] + jnp.sum(probs, axis=0, keepdims=True)

    @pl.when(i == NB_PRE - 1)
    def _():
        f = facc[...] / float(N)
        pmean = pacc[...] / float(N)
        bal_ref[0, 0] = jnp.sum(f * pmean) * float(E)


def _tc_pre(emb_rows, pos, typ, lng, lnb, mapw, mapb, gatew, gateb):
    full2 = lambda shape: pl.BlockSpec(shape, lambda i: (0, 0))
    return pl.pallas_call(
        _tc_pre_body,
        grid=(NB_PRE,),
        in_specs=[
            pl.BlockSpec((TB_PRE, EMB), lambda i: (i, 0)),
            full2((S, EMB)),
            full2((1, EMB)),
            full2((1, EMB)),
            full2((1, EMB)),
            full2((EMB, D)),
            full2((1, D)),
            full2((D, E)),
            full2((1, E)),
        ],
        out_specs=[
            pl.BlockSpec((TB_PRE, D), lambda i: (i, 0)),
            pl.BlockSpec((TB_PRE, 1), lambda i: (i, 0)),
            pl.BlockSpec((TB_PRE, 1), lambda i: (i, 0)),
            pl.BlockSpec((TB_PRE, 1), lambda i: (i, 0)),
            pl.BlockSpec((TB_PRE, 1), lambda i: (i, 0)),
            pl.BlockSpec(memory_space=pltpu.SMEM),
        ],
        out_shape=[
            jax.ShapeDtypeStruct((N, D), F32),
            jax.ShapeDtypeStruct((N, 1), I32),
            jax.ShapeDtypeStruct((N, 1), I32),
            jax.ShapeDtypeStruct((N, 1), F32),
            jax.ShapeDtypeStruct((N, 1), F32),
            jax.ShapeDtypeStruct((1, 1), F32),
        ],
        scratch_shapes=[
            pltpu.VMEM((1, E), F32),
            pltpu.VMEM((1, E), F32),
            pltpu.VMEM((1, E), F32),
        ],
    )(emb_rows, pos, typ, lng, lnb, mapw, mapb, gatew, gateb)


def _tc_experts_body(disp_ref, w1_ref, b1_ref, w2_ref, b2_ref, out_ref):
    d = disp_ref[...].astype(jnp.bfloat16)
    h = jnp.dot(d, w1_ref[0].astype(jnp.bfloat16),
                preferred_element_type=F32) + b1_ref[0]
    h = jax.nn.gelu(h)
    out = jnp.dot(h.astype(jnp.bfloat16), w2_ref[0].astype(jnp.bfloat16),
                  preferred_element_type=F32) + b2_ref[0]
    out_ref[...] = out


def _tc_experts(disp, w1, b1, w2, b2):
    return pl.pallas_call(
        _tc_experts_body,
        grid=(E,),
        in_specs=[
            pl.BlockSpec((CAP, D), lambda e: (e, 0)),
            pl.BlockSpec((1, D, FFN), lambda e: (e, 0, 0)),
            pl.BlockSpec((1, 1, FFN), lambda e: (e, 0, 0)),
            pl.BlockSpec((1, FFN, D), lambda e: (e, 0, 0)),
            pl.BlockSpec((1, 1, D), lambda e: (e, 0, 0)),
        ],
        out_specs=pl.BlockSpec((CAP, D), lambda e: (e, 0)),
        out_shape=jax.ShapeDtypeStruct((E * CAP, D), F32),
    )(disp, w1, b1.reshape(E, 1, FFN), w2, b2.reshape(E, 1, D))


TB_MLM = 64
NB_MLM = N // TB_MLM
BPB = S // TB_MLM


def _tc_mlm_body(y_ref, p_ref, keep_ref, lbl_ref, mlmw_ref, mlmb_ref,
                 lng_ref, lnb_ref, decw_ref, decb_ref, bal_ref,
                 h_ref, sc_ref, loss_ref, lacc):
    i = pl.program_id(0)

    @pl.when(i == 0)
    def _():
        lacc[0, 0] = 0.0

    y = jnp.where(keep_ref[...] > 0.0, y_ref[...] * p_ref[...], 0.0)
    h_ref[...] = y

    m = jnp.dot(y, mlmw_ref[...], preferred_element_type=F32) + mlmb_ref[...]
    m = jax.nn.gelu(m)
    mu = jnp.mean(m, axis=-1, keepdims=True)
    var = jnp.mean((m - mu) ** 2, axis=-1, keepdims=True)
    mln = (m - mu) / jnp.sqrt(var + EPS) * lng_ref[...] + lnb_ref[...]

    sc = jnp.dot(mln.astype(jnp.bfloat16), decw_ref[...],
                 preferred_element_type=F32) + decb_ref[...]
    sc_ref[...] = sc

    mx = jnp.max(sc, axis=-1, keepdims=True)
    lse = mx + jnp.log(jnp.sum(jnp.exp(sc - mx), axis=-1, keepdims=True))
    vid = lax.broadcasted_iota(I32, (TB_MLM, VOCAB), 1)
    slab = jnp.sum(jnp.where(vid == lbl_ref[...], sc, 0.0), axis=-1,
                   keepdims=True)
    lacc[0, 0] += jnp.sum(lse - slab)

    @pl.when(i == NB_MLM - 1)
    def _():
        loss_ref[0, 0] = lacc[0, 0] / float(N) + bal_ref[0, 0]


def _tc_mlm(yraw, p, keep, lbl, mlmw, mlmb, lng, lnb, decw, decb, bal):
    full2 = lambda shape: pl.BlockSpec(shape, lambda i: (0, 0))
    tok = lambda: pl.BlockSpec((TB_MLM, 1), lambda i: (i, 0))
    return pl.pallas_call(
        _tc_mlm_body,
        grid=(NB_MLM,),
        in_specs=[
            pl.BlockSpec((TB_MLM, D), lambda i: (i, 0)),
            tok(),
            tok(),
            tok(),
            full2((D, EMB)),
            full2((1, EMB)),
            full2((1, EMB)),
            full2((1, EMB)),
            full2((EMB, VOCAB)),
            full2((1, VOCAB)),
            pl.BlockSpec(memory_space=pltpu.SMEM),
        ],
        out_specs=[
            pl.BlockSpec((TB_MLM, D), lambda i: (i, 0)),
            pl.BlockSpec((TB_MLM, VOCAB), lambda i: (i, 0)),
            pl.BlockSpec(memory_space=pltpu.SMEM),
        ],
        out_shape=[
            jax.ShapeDtypeStruct((N, D), F32),
            jax.ShapeDtypeStruct((N, VOCAB), F32),
            jax.ShapeDtypeStruct((1, 1), F32),
        ],
        scratch_shapes=[pltpu.SMEM((1, 1), F32)],
    )(yraw, p, keep, lbl, mlmw, mlmb, lng, lnb, decw, decb, bal)


def kernel(input_ids, labels, word_emb, pos_emb, type_emb, emb_ln_g, emb_ln_b,
           map_W, map_b, gate_W, gate_b, w1, b1, w2, b2,
           mlm_W, mlm_b, mlm_ln_g, mlm_ln_b, dec_W, dec_b):
    ids = input_ids.reshape(-1).astype(I32)

    emb_rows = _sc_embed_gather(word_emb, ids)

    x, dest, gidx, pval, keep, bal = _tc_pre(
        emb_rows, pos_emb[:S], type_emb[0:1],
        emb_ln_g.reshape(1, EMB), emb_ln_b.reshape(1, EMB),
        map_W, map_b.reshape(1, D), gate_W, gate_b.reshape(1, E))

    disp = _sc_dispatch(x, dest.reshape(-1))
    eout = _tc_experts(disp, w1, b1, w2, b2)
    yraw = _sc_combine(eout, gidx.reshape(-1))

    h, scores, loss = _tc_mlm(
        yraw, pval, keep, labels.reshape(-1, 1).astype(I32),
        mlm_W, mlm_b.reshape(1, EMB),
        mlm_ln_g.reshape(1, EMB), mlm_ln_b.reshape(1, EMB),
        dec_W.astype(jnp.bfloat16), dec_b.reshape(1, VOCAB), bal)

    return (loss[0, 0], scores.reshape(B, S, VOCAB), h.reshape(B, S, D))
